# baseline (speedup 1.0000x reference)
import functools

import jax
import jax.numpy as jnp
from jax import lax
from jax.experimental import pallas as pl
from jax.experimental.pallas import tpu as pltpu
from jax.experimental.pallas import tpu_sc as plsc

N_ROWS = 16384
NC = 2
NS = 16
L = 16
NW = NC * NS
ROWS_W = N_ROWS // NW
STEPS = ROWS_W // L


def _sc_body(o_hbm, t_hbm, l_hbm, sums_hbm, cnts_hbm,
             o_v, t_v, l_v, acc_v, cnt_v, sem):
    cid = lax.axis_index("c")
    sid = lax.axis_index("s")
    w = sid * NC + cid
    base_r = w * ROWS_W

    copies = [pltpu.async_copy(l_hbm.at[pl.ds(base_r, ROWS_W)], l_v, sem)]
    for c in range(4):
        copies.append(pltpu.async_copy(
            o_hbm.at[c, pl.ds(base_r, ROWS_W)],
            o_v.at[pl.ds(c * ROWS_W, ROWS_W)], sem))
        copies.append(pltpu.async_copy(
            t_hbm.at[c, pl.ds(base_r, ROWS_W)],
            t_v.at[pl.ds(c * ROWS_W, ROWS_W)], sem))
    for cp in copies:
        cp.wait()

    ones = jnp.ones((L,), jnp.float32)
    zeros = jnp.zeros((L,), jnp.float32)

    def step(i, carry):
        a0, a1, a2, a3, cnt = carry
        base = i * L
        lab = l_v[pl.ds(base, L)]
        validf = jnp.where(jnp.abs(lab) == 1, ones, zeros)
        d0 = o_v[pl.ds(base, L)] - t_v[pl.ds(base, L)]
        d1 = o_v[pl.ds(ROWS_W + base, L)] - t_v[pl.ds(ROWS_W + base, L)]
        d2 = o_v[pl.ds(2 * ROWS_W + base, L)] - t_v[pl.ds(2 * ROWS_W + base, L)]
        d3 = o_v[pl.ds(3 * ROWS_W + base, L)] - t_v[pl.ds(3 * ROWS_W + base, L)]
        return (a0 + validf * (d0 * d0),
                a1 + validf * (d1 * d1),
                a2 + validf * (d2 * d2),
                a3 + validf * (d3 * d3),
                cnt + validf)

    a0, a1, a2, a3, cnt = lax.fori_loop(
        0, STEPS, step, (zeros, zeros, zeros, zeros, zeros))
    acc_v[...] = (a0 + a1) + (a2 + a3)
    cnt_v[...] = cnt
    pltpu.sync_copy(acc_v, sums_hbm.at[pl.ds(w * L, L)])
    pltpu.sync_copy(cnt_v, cnts_hbm.at[pl.ds(w * L, L)])


_sc_call = functools.partial(
    pl.kernel,
    out_type=(jax.ShapeDtypeStruct((NW * L,), jnp.float32),
              jax.ShapeDtypeStruct((NW * L,), jnp.float32)),
    mesh=plsc.VectorSubcoreMesh(core_axis_name="c", subcore_axis_name="s",
                                num_cores=NC, num_subcores=NS),
    scratch_types=[
        pltpu.VMEM((4 * ROWS_W,), jnp.float32),
        pltpu.VMEM((4 * ROWS_W,), jnp.float32),
        pltpu.VMEM((ROWS_W,), jnp.int32),
        pltpu.VMEM((L,), jnp.float32),
        pltpu.VMEM((L,), jnp.float32),
        pltpu.SemaphoreType.DMA,
    ],
)(_sc_body)


def kernel(bbox_out, bbox_target, label):
    sums, cnts = _sc_call(bbox_out.T, bbox_target.T, label)
    total = jnp.sum(sums)
    keep_num = jnp.sum(cnts)
    return total / keep_num
